# Initial kernel scaffold; baseline (speedup 1.0000x reference)
import jax
import jax.numpy as jnp
from jax.experimental import pallas as pl


def kernel(x, edge_index, W1_rel, b1, W1_root, W2_rel, b2, W2_root, Wc, bc):
    raise NotImplementedError("write your pallas kernel here")


# speedup vs baseline: 5.3690x; 5.3690x over previous
import functools

import jax
import jax.numpy as jnp
from jax import lax
from jax.experimental import pallas as pl
from jax.experimental.pallas import tpu as pltpu
from jax.experimental.pallas import tpu_sc as plsc

N = 10000
D = 128
H = 64
E = 320000
CHUNK = 128
N_PAD = N + 16
N_AGG = 10240


def _tc1_body(x_ref, wrel_ref, wroot_ref, b_ref, y_ref, z_ref):
    x = x_ref[...]
    y_ref[0:N, :] = jnp.dot(x, wrel_ref[...], preferred_element_type=jnp.float32)
    y_ref[N:N_PAD, :] = jnp.zeros((N_PAD - N, H), jnp.float32)
    z_ref[...] = (
        jnp.dot(x, wroot_ref[...], preferred_element_type=jnp.float32) + b_ref[...]
    )


def _tc2_body(aggp_ref, z1_ref, wrel_ref, wroot_ref, b_ref, y_ref, z_ref):
    agg = aggp_ref[0, 0:N, :] + aggp_ref[1, 0:N, :]
    h1 = jnp.maximum(agg + z1_ref[...], 0.0)
    y_ref[0:N, :] = jnp.dot(h1, wrel_ref[...], preferred_element_type=jnp.float32)
    y_ref[N:N_PAD, :] = jnp.zeros((N_PAD - N, H), jnp.float32)
    z_ref[...] = (
        jnp.dot(h1, wroot_ref[...], preferred_element_type=jnp.float32) + b_ref[...]
    )


def _tc3_body(aggp_ref, z2_ref, wc_ref, bc_ref, out_ref):
    h2 = jnp.maximum(aggp_ref[0, 0:N, :] + aggp_ref[1, 0:N, :] + z2_ref[...], 0.0)
    pooled = jnp.sum(h2, axis=0, keepdims=True) * (1.0 / N)
    out_ref[...] = (
        jnp.dot(pooled, wc_ref[...], preferred_element_type=jnp.float32) + bc_ref[...]
    )


def _make_sc_scatter(nc, ns, k_per_tile):
    nw = nc * ns
    rows_per_tile = N_AGG // ns

    mesh = plsc.VectorSubcoreMesh(core_axis_name="c", subcore_axis_name="s")

    @functools.partial(
        pl.kernel,
        mesh=mesh,
        out_type=jax.ShapeDtypeStruct((2, N_AGG, H), jnp.float32),
        scratch_types=[
            pltpu.VMEM((k_per_tile, CHUNK), jnp.int32),
            pltpu.VMEM((k_per_tile, CHUNK), jnp.int32),
            pltpu.VMEM((CHUNK, H), jnp.float32),
            pltpu.VMEM_SHARED((N_AGG, H), jnp.float32),
            pltpu.SemaphoreType.DMA,
        ],
        compiler_params=pltpu.CompilerParams(use_tc_tiling_on_sc=False),
    )
    def sc_scatter(y_hbm, src_hbm, dst_hbm, zeros_hbm, out_hbm,
                   src_v, dst_v, rows_v, agg_sh, sem):
        c = lax.axis_index("c")
        s = lax.axis_index("s")
        w = s * nc + c

        pltpu.sync_copy(zeros_hbm, agg_sh.at[pl.ds(s * rows_per_tile, rows_per_tile)])
        pltpu.sync_copy(src_hbm.at[w], src_v)
        pltpu.sync_copy(dst_hbm.at[w], dst_v)
        plsc.subcore_barrier()

        def step(j, carry):
            pltpu.async_copy(y_hbm.at[src_v.at[j]], rows_v, sem).wait()
            pltpu.sync_copy(rows_v, agg_sh.at[dst_v.at[j]], add=True)
            return carry

        lax.fori_loop(0, k_per_tile, step, 0)
        plsc.subcore_barrier()

        sl = pl.ds(s * rows_per_tile, rows_per_tile)
        pltpu.sync_copy(agg_sh.at[sl], out_hbm.at[c].at[sl])

    return sc_scatter


def kernel(x, edge_index, W1_rel, b1, W1_root, W2_rel, b2, W2_root, Wc, bc):
    info = plsc.get_sparse_core_info()
    nc, ns = info.num_cores, info.num_subcores
    nw = nc * ns

    k_per_tile = -(-E // (nw * CHUNK))
    k_per_tile = -(-k_per_tile // 8) * 8
    e_pad = nw * k_per_tile * CHUNK

    src = edge_index[0]
    dst = edge_index[1]
    src_p = jnp.concatenate(
        [src, jnp.full((e_pad - E,), N, jnp.int32)]).reshape(nw, k_per_tile, CHUNK)
    dst_p = jnp.concatenate(
        [dst, jnp.zeros((e_pad - E,), jnp.int32)]).reshape(nw, k_per_tile, CHUNK)
    zeros_tile = jnp.zeros((N_AGG // ns, H), jnp.float32)

    sc_scatter = _make_sc_scatter(nc, ns, k_per_tile)

    tc1 = pl.pallas_call(
        _tc1_body,
        out_shape=[
            jax.ShapeDtypeStruct((N_PAD, H), jnp.float32),
            jax.ShapeDtypeStruct((N, H), jnp.float32),
        ],
    )
    y1, z1 = tc1(x, W1_rel, W1_root, b1.reshape(1, H))

    agg1 = sc_scatter(y1, src_p, dst_p, zeros_tile)

    tc2 = pl.pallas_call(
        _tc2_body,
        out_shape=[
            jax.ShapeDtypeStruct((N_PAD, H), jnp.float32),
            jax.ShapeDtypeStruct((N, H), jnp.float32),
        ],
    )
    y2, z2 = tc2(agg1, z1, W2_rel, W2_root, b2.reshape(1, H))

    agg2 = sc_scatter(y2, src_p, dst_p, zeros_tile)

    tc3 = pl.pallas_call(
        _tc3_body,
        out_shape=jax.ShapeDtypeStruct((1, 2), jnp.float32),
    )
    out = tc3(agg2, z2, Wc, bc.reshape(1, 2))
    return out.reshape(2)


# speedup vs baseline: 5.7765x; 1.0759x over previous
import functools

import jax
import jax.numpy as jnp
from jax import lax
from jax.experimental import pallas as pl
from jax.experimental.pallas import tpu as pltpu
from jax.experimental.pallas import tpu_sc as plsc

N = 10000
D = 128
H = 64
E = 320000
CHUNK = 128
N_PAD = N + 16
N_AGG = 10240


def _tc1_body(x_ref, wrel_ref, wroot_ref, b_ref, y_ref, z_ref):
    x = x_ref[...]
    y_ref[0:N, :] = jnp.dot(x, wrel_ref[...], preferred_element_type=jnp.float32)
    y_ref[N:N_PAD, :] = jnp.zeros((N_PAD - N, H), jnp.float32)
    z_ref[...] = (
        jnp.dot(x, wroot_ref[...], preferred_element_type=jnp.float32) + b_ref[...]
    )


def _tc2_body(aggp_ref, z1_ref, wrel_ref, wroot_ref, b_ref, y_ref, z_ref):
    agg = aggp_ref[0, 0:N, :] + aggp_ref[1, 0:N, :]
    h1 = jnp.maximum(agg + z1_ref[...], 0.0)
    y_ref[0:N, :] = jnp.dot(h1, wrel_ref[...], preferred_element_type=jnp.float32)
    y_ref[N:N_PAD, :] = jnp.zeros((N_PAD - N, H), jnp.float32)
    z_ref[...] = (
        jnp.dot(h1, wroot_ref[...], preferred_element_type=jnp.float32) + b_ref[...]
    )


def _tc3_body(aggp_ref, z2_ref, wc_ref, bc_ref, out_ref):
    h2 = jnp.maximum(aggp_ref[0, 0:N, :] + aggp_ref[1, 0:N, :] + z2_ref[...], 0.0)
    pooled = jnp.sum(h2, axis=0, keepdims=True) * (1.0 / N)
    out_ref[...] = (
        jnp.dot(pooled, wc_ref[...], preferred_element_type=jnp.float32) + bc_ref[...]
    )


def _make_sc_scatter(nc, ns, k_per_tile):
    nw = nc * ns
    rows_per_tile = N_AGG // ns

    mesh = plsc.VectorSubcoreMesh(core_axis_name="c", subcore_axis_name="s")

    @functools.partial(
        pl.kernel,
        mesh=mesh,
        out_type=jax.ShapeDtypeStruct((2, N_AGG, H), jnp.float32),
        scratch_types=[
            pltpu.VMEM((k_per_tile, CHUNK), jnp.int32),
            pltpu.VMEM((k_per_tile, CHUNK), jnp.int32),
            pltpu.VMEM((2, CHUNK, H), jnp.float32),
            pltpu.VMEM_SHARED((N_AGG, H), jnp.float32),
            pltpu.SemaphoreType.DMA,
            pltpu.SemaphoreType.DMA,
        ],
        compiler_params=pltpu.CompilerParams(use_tc_tiling_on_sc=False),
    )
    def sc_scatter(y_hbm, src_hbm, dst_hbm, zeros_hbm, out_hbm,
                   src_v, dst_v, rows_v, agg_sh, sem0, sem1):
        c = lax.axis_index("c")
        s = lax.axis_index("s")
        w = s * nc + c

        pltpu.sync_copy(zeros_hbm, agg_sh.at[pl.ds(s * rows_per_tile, rows_per_tile)])
        pltpu.sync_copy(src_hbm.at[w], src_v)
        pltpu.sync_copy(dst_hbm.at[w], dst_v)
        plsc.subcore_barrier()

        def gather(j, b, sem):
            pltpu.async_copy(y_hbm.at[src_v.at[j]], rows_v.at[b], sem)

        def gwait(b, sem):
            pltpu.make_async_copy(y_hbm.at[src_v.at[0]], rows_v.at[b], sem).wait()

        def scatter(j, b):
            pltpu.sync_copy(rows_v.at[b], agg_sh.at[dst_v.at[j]], add=True)

        gather(0, 0, sem0)

        def group(g, carry):
            j0 = 2 * g
            j1 = j0 + 1
            jn = jnp.minimum(j0 + 2, k_per_tile - 1)
            gwait(0, sem0)
            gather(j1, 1, sem1)
            scatter(j0, 0)
            gwait(1, sem1)
            gather(jn, 0, sem0)
            scatter(j1, 1)
            return carry

        lax.fori_loop(0, k_per_tile // 2, group, 0)
        gwait(0, sem0)
        plsc.subcore_barrier()

        sl = pl.ds(s * rows_per_tile, rows_per_tile)
        pltpu.sync_copy(agg_sh.at[sl], out_hbm.at[c].at[sl])

    return sc_scatter


def kernel(x, edge_index, W1_rel, b1, W1_root, W2_rel, b2, W2_root, Wc, bc):
    info = plsc.get_sparse_core_info()
    nc, ns = info.num_cores, info.num_subcores
    nw = nc * ns

    k_per_tile = -(-E // (nw * CHUNK))
    k_per_tile = -(-k_per_tile // 8) * 8
    e_pad = nw * k_per_tile * CHUNK

    src = edge_index[0]
    dst = edge_index[1]
    src_p = jnp.concatenate(
        [src, jnp.full((e_pad - E,), N, jnp.int32)]).reshape(nw, k_per_tile, CHUNK)
    dst_p = jnp.concatenate(
        [dst, jnp.zeros((e_pad - E,), jnp.int32)]).reshape(nw, k_per_tile, CHUNK)
    zeros_tile = jnp.zeros((N_AGG // ns, H), jnp.float32)

    sc_scatter = _make_sc_scatter(nc, ns, k_per_tile)

    tc1 = pl.pallas_call(
        _tc1_body,
        out_shape=[
            jax.ShapeDtypeStruct((N_PAD, H), jnp.float32),
            jax.ShapeDtypeStruct((N, H), jnp.float32),
        ],
    )
    y1, z1 = tc1(x, W1_rel, W1_root, b1.reshape(1, H))

    agg1 = sc_scatter(y1, src_p, dst_p, zeros_tile)

    tc2 = pl.pallas_call(
        _tc2_body,
        out_shape=[
            jax.ShapeDtypeStruct((N_PAD, H), jnp.float32),
            jax.ShapeDtypeStruct((N, H), jnp.float32),
        ],
    )
    y2, z2 = tc2(agg1, z1, W2_rel, W2_root, b2.reshape(1, H))

    agg2 = sc_scatter(y2, src_p, dst_p, zeros_tile)

    tc3 = pl.pallas_call(
        _tc3_body,
        out_shape=jax.ShapeDtypeStruct((1, 2), jnp.float32),
    )
    out = tc3(agg2, z2, Wc, bc.reshape(1, 2))
    return out.reshape(2)


# speedup vs baseline: 13.3184x; 2.3056x over previous
import functools

import jax
import jax.numpy as jnp
from jax import lax
from jax.experimental import pallas as pl
from jax.experimental.pallas import tpu as pltpu
from jax.experimental.pallas import tpu_sc as plsc

N = 10000
D = 128
H = 64
E = 320000
CHUNK = 128
N_PAD = 10240
N_AGG = N_PAD


def _tc1_body(x_ref, wrel_ref, wroot_ref, b_ref, y_ref, z_ref):
    x = x_ref[...]
    y_ref[0:N, :] = jnp.dot(x, wrel_ref[...], preferred_element_type=jnp.float32)
    y_ref[N:N_PAD, :] = jnp.zeros((N_PAD - N, H), jnp.float32)
    z_ref[...] = (
        jnp.dot(x, wroot_ref[...], preferred_element_type=jnp.float32) + b_ref[...]
    )


def _tc2_body(aggp_ref, z1_ref, wrel_ref, wroot_ref, b_ref, y_ref, z_ref):
    agg = aggp_ref[0, 0:N, :] + aggp_ref[1, 0:N, :]
    h1 = jnp.maximum(agg + z1_ref[...], 0.0)
    y_ref[0:N, :] = jnp.dot(h1, wrel_ref[...], preferred_element_type=jnp.float32)
    y_ref[N:N_PAD, :] = jnp.zeros((N_PAD - N, H), jnp.float32)
    z_ref[...] = (
        jnp.dot(h1, wroot_ref[...], preferred_element_type=jnp.float32) + b_ref[...]
    )


def _tc3_body(aggp_ref, z2_ref, wc_ref, bc_ref, out_ref):
    h2 = jnp.maximum(aggp_ref[0, 0:N, :] + aggp_ref[1, 0:N, :] + z2_ref[...], 0.0)
    pooled = jnp.sum(h2, axis=0, keepdims=True) * (1.0 / N)
    out_ref[...] = (
        jnp.dot(pooled, wc_ref[...], preferred_element_type=jnp.float32) + bc_ref[...]
    )


def _make_sc_scatter(nc, ns, k_per_tile):
    nw = nc * ns
    rows_per_tile = N_AGG // ns

    mesh = plsc.VectorSubcoreMesh(core_axis_name="c", subcore_axis_name="s")

    @functools.partial(
        pl.kernel,
        mesh=mesh,
        out_type=jax.ShapeDtypeStruct((2, N_AGG, H), jnp.float32),
        scratch_types=[
            pltpu.VMEM((k_per_tile, CHUNK), jnp.int32),
            pltpu.VMEM((k_per_tile, CHUNK), jnp.int32),
            pltpu.VMEM((2, CHUNK, H), jnp.float32),
            pltpu.VMEM_SHARED((N_AGG, H), jnp.float32),
            pltpu.VMEM_SHARED((N_PAD, H), jnp.float32),
            pltpu.SemaphoreType.DMA,
            pltpu.SemaphoreType.DMA,
        ],
        compiler_params=pltpu.CompilerParams(use_tc_tiling_on_sc=False),
    )
    def sc_scatter(y_hbm, src_hbm, dst_hbm, zeros_hbm, out_hbm,
                   src_v, dst_v, rows_v, agg_sh, y_sh, sem0, sem1):
        c = lax.axis_index("c")
        s = lax.axis_index("s")
        w = s * nc + c

        sl = pl.ds(s * rows_per_tile, rows_per_tile)
        pltpu.sync_copy(zeros_hbm, agg_sh.at[sl])
        pltpu.sync_copy(y_hbm.at[sl], y_sh.at[sl])
        pltpu.sync_copy(src_hbm.at[w], src_v)
        pltpu.sync_copy(dst_hbm.at[w], dst_v)
        plsc.subcore_barrier()

        def gather(j, b, sem):
            pltpu.async_copy(y_sh.at[src_v.at[j]], rows_v.at[b], sem)

        def gwait(b, sem):
            pltpu.make_async_copy(y_sh.at[src_v.at[0]], rows_v.at[b], sem).wait()

        def scatter(j, b):
            pltpu.sync_copy(rows_v.at[b], agg_sh.at[dst_v.at[j]], add=True)

        gather(0, 0, sem0)

        def group(g, carry):
            j0 = 2 * g
            j1 = j0 + 1
            jn = jnp.minimum(j0 + 2, k_per_tile - 1)
            gwait(0, sem0)
            gather(j1, 1, sem1)
            scatter(j0, 0)
            gwait(1, sem1)
            gather(jn, 0, sem0)
            scatter(j1, 1)
            return carry

        lax.fori_loop(0, k_per_tile // 2, group, 0)
        gwait(0, sem0)
        plsc.subcore_barrier()

        pltpu.sync_copy(agg_sh.at[sl], out_hbm.at[c].at[sl])

    return sc_scatter


def kernel(x, edge_index, W1_rel, b1, W1_root, W2_rel, b2, W2_root, Wc, bc):
    info = plsc.get_sparse_core_info()
    nc, ns = info.num_cores, info.num_subcores
    nw = nc * ns

    k_per_tile = -(-E // (nw * CHUNK))
    k_per_tile = -(-k_per_tile // 8) * 8
    e_pad = nw * k_per_tile * CHUNK

    src = edge_index[0]
    dst = edge_index[1]
    src_p = jnp.concatenate(
        [src, jnp.full((e_pad - E,), N, jnp.int32)]).reshape(nw, k_per_tile, CHUNK)
    dst_p = jnp.concatenate(
        [dst, jnp.zeros((e_pad - E,), jnp.int32)]).reshape(nw, k_per_tile, CHUNK)
    zeros_tile = jnp.zeros((N_AGG // ns, H), jnp.float32)

    sc_scatter = _make_sc_scatter(nc, ns, k_per_tile)

    tc1 = pl.pallas_call(
        _tc1_body,
        out_shape=[
            jax.ShapeDtypeStruct((N_PAD, H), jnp.float32),
            jax.ShapeDtypeStruct((N, H), jnp.float32),
        ],
    )
    y1, z1 = tc1(x, W1_rel, W1_root, b1.reshape(1, H))

    agg1 = sc_scatter(y1, src_p, dst_p, zeros_tile)

    tc2 = pl.pallas_call(
        _tc2_body,
        out_shape=[
            jax.ShapeDtypeStruct((N_PAD, H), jnp.float32),
            jax.ShapeDtypeStruct((N, H), jnp.float32),
        ],
    )
    y2, z2 = tc2(agg1, z1, W2_rel, W2_root, b2.reshape(1, H))

    agg2 = sc_scatter(y2, src_p, dst_p, zeros_tile)

    tc3 = pl.pallas_call(
        _tc3_body,
        out_shape=jax.ShapeDtypeStruct((1, 2), jnp.float32),
    )
    out = tc3(agg2, z2, Wc, bc.reshape(1, 2))
    return out.reshape(2)


# speedup vs baseline: 13.3576x; 1.0029x over previous
import functools

import jax
import jax.numpy as jnp
from jax import lax
from jax.experimental import pallas as pl
from jax.experimental.pallas import tpu as pltpu
from jax.experimental.pallas import tpu_sc as plsc

N = 10000
D = 128
H = 64
E = 320000
CHUNK = 128
N_PAD = 10240
N_AGG = N_PAD


def _tc1_body(x_ref, wrel_ref, wroot_ref, b_ref, y_ref, z_ref):
    x = x_ref[...]
    y_ref[0:N, :] = jnp.dot(x, wrel_ref[...], preferred_element_type=jnp.float32)
    y_ref[N:N_PAD, :] = jnp.zeros((N_PAD - N, H), jnp.float32)
    z_ref[...] = (
        jnp.dot(x, wroot_ref[...], preferred_element_type=jnp.float32) + b_ref[...]
    )


def _tc2_body(aggp_ref, z1_ref, wrel_ref, wroot_ref, b_ref, y_ref, z_ref):
    agg = aggp_ref[0, 0:N, :] + aggp_ref[1, 0:N, :]
    h1 = jnp.maximum(agg + z1_ref[...], 0.0)
    y_ref[0:N, :] = jnp.dot(h1, wrel_ref[...], preferred_element_type=jnp.float32)
    y_ref[N:N_PAD, :] = jnp.zeros((N_PAD - N, H), jnp.float32)
    z_ref[...] = (
        jnp.dot(h1, wroot_ref[...], preferred_element_type=jnp.float32) + b_ref[...]
    )


def _tc3_body(aggp_ref, z2_ref, wc_ref, bc_ref, out_ref):
    h2 = jnp.maximum(aggp_ref[0, 0:N, :] + aggp_ref[1, 0:N, :] + z2_ref[...], 0.0)
    pooled = jnp.sum(h2, axis=0, keepdims=True) * (1.0 / N)
    out_ref[...] = (
        jnp.dot(pooled, wc_ref[...], preferred_element_type=jnp.float32) + bc_ref[...]
    )


def _make_sc_scatter(nc, ns, k_per_tile):
    nw = nc * ns
    rows_per_tile = N_AGG // ns

    mesh = plsc.VectorSubcoreMesh(core_axis_name="c", subcore_axis_name="s")

    @functools.partial(
        pl.kernel,
        mesh=mesh,
        out_type=jax.ShapeDtypeStruct((2, N_AGG, H), jnp.float32),
        scratch_types=[
            pltpu.VMEM((k_per_tile, CHUNK), jnp.int32),
            pltpu.VMEM((k_per_tile, CHUNK), jnp.int32),
            pltpu.VMEM((2, CHUNK, H), jnp.float32),
            pltpu.VMEM_SHARED((N_AGG, H), jnp.float32),
            pltpu.VMEM_SHARED((N_PAD, H), jnp.float32),
            pltpu.SemaphoreType.DMA,
            pltpu.SemaphoreType.DMA,
            pltpu.SemaphoreType.DMA,
            pltpu.SemaphoreType.DMA,
        ],
        compiler_params=pltpu.CompilerParams(use_tc_tiling_on_sc=False),
    )
    def sc_scatter(y_hbm, src_hbm, dst_hbm, zeros_hbm, out_hbm,
                   src_v, dst_v, rows_v, agg_sh, y_sh, sem0, sem1, ssem0, ssem1):
        c = lax.axis_index("c")
        s = lax.axis_index("s")
        w = s * nc + c

        sl = pl.ds(s * rows_per_tile, rows_per_tile)
        pltpu.sync_copy(zeros_hbm, agg_sh.at[sl])
        pltpu.sync_copy(y_hbm.at[sl], y_sh.at[sl])
        pltpu.sync_copy(src_hbm.at[w], src_v)
        pltpu.sync_copy(dst_hbm.at[w], dst_v)
        plsc.subcore_barrier()

        def gather(j, b, sem):
            pltpu.async_copy(y_sh.at[src_v.at[j]], rows_v.at[b], sem)

        def gwait(b, sem):
            pltpu.make_async_copy(y_sh.at[src_v.at[0]], rows_v.at[b], sem).wait()

        def scatter(j, b, sem):
            pltpu.async_copy(rows_v.at[b], agg_sh.at[dst_v.at[j]], sem, add=True)

        def swait(b, sem):
            pltpu.make_async_copy(rows_v.at[b], agg_sh.at[dst_v.at[0]], sem).wait()

        gather(0, 0, sem0)

        def group(g, carry):
            j0 = 2 * g
            j1 = j0 + 1
            jn = jnp.minimum(j0 + 2, k_per_tile - 1)
            gwait(0, sem0)

            @pl.when(g > 0)
            def _():
                swait(1, ssem1)

            gather(j1, 1, sem1)
            scatter(j0, 0, ssem0)
            gwait(1, sem1)
            swait(0, ssem0)
            gather(jn, 0, sem0)
            scatter(j1, 1, ssem1)
            return carry

        lax.fori_loop(0, k_per_tile // 2, group, 0)
        gwait(0, sem0)
        swait(1, ssem1)
        plsc.subcore_barrier()

        pltpu.sync_copy(agg_sh.at[sl], out_hbm.at[c].at[sl])

    return sc_scatter


def kernel(x, edge_index, W1_rel, b1, W1_root, W2_rel, b2, W2_root, Wc, bc):
    info = plsc.get_sparse_core_info()
    nc, ns = info.num_cores, info.num_subcores
    nw = nc * ns

    k_per_tile = -(-E // (nw * CHUNK))
    k_per_tile = -(-k_per_tile // 8) * 8
    e_pad = nw * k_per_tile * CHUNK

    src = edge_index[0]
    dst = edge_index[1]
    src_p = jnp.concatenate(
        [src, jnp.full((e_pad - E,), N, jnp.int32)]).reshape(nw, k_per_tile, CHUNK)
    dst_p = jnp.concatenate(
        [dst, jnp.zeros((e_pad - E,), jnp.int32)]).reshape(nw, k_per_tile, CHUNK)
    zeros_tile = jnp.zeros((N_AGG // ns, H), jnp.float32)

    sc_scatter = _make_sc_scatter(nc, ns, k_per_tile)

    tc1 = pl.pallas_call(
        _tc1_body,
        out_shape=[
            jax.ShapeDtypeStruct((N_PAD, H), jnp.float32),
            jax.ShapeDtypeStruct((N, H), jnp.float32),
        ],
    )
    y1, z1 = tc1(x, W1_rel, W1_root, b1.reshape(1, H))

    agg1 = sc_scatter(y1, src_p, dst_p, zeros_tile)

    tc2 = pl.pallas_call(
        _tc2_body,
        out_shape=[
            jax.ShapeDtypeStruct((N_PAD, H), jnp.float32),
            jax.ShapeDtypeStruct((N, H), jnp.float32),
        ],
    )
    y2, z2 = tc2(agg1, z1, W2_rel, W2_root, b2.reshape(1, H))

    agg2 = sc_scatter(y2, src_p, dst_p, zeros_tile)

    tc3 = pl.pallas_call(
        _tc3_body,
        out_shape=jax.ShapeDtypeStruct((1, 2), jnp.float32),
    )
    out = tc3(agg2, z2, Wc, bc.reshape(1, 2))
    return out.reshape(2)
